# Initial kernel scaffold; baseline (speedup 1.0000x reference)
import jax
import jax.numpy as jnp
from jax.experimental import pallas as pl


def kernel(cls_score, label):
    raise NotImplementedError("write your pallas kernel here")


# speedup vs baseline: 12.1445x; 12.1445x over previous
import jax
import jax.numpy as jnp
import numpy as np
from jax.experimental import pallas as pl

_B = 8000
_C = 27
_GROUPS = ((1, 14), (15, 10), (25, 2))


def _partials_body(x_ref, lb_ref, out_ref):
    i = pl.program_id(0)
    x = x_ref[...]
    lb = lb_ref[0]
    f32 = jnp.float32
    B = x.shape[0]

    fg0m = (lb >= 1) & (lb <= 13)
    fg1m = lb >= 14
    w0 = (lb != 22).astype(f32)
    col0 = jnp.where(fg0m, lb + 1, 1)
    col1 = jnp.where(fg1m, lb + 2, 15)
    col2 = jnp.where(fg1m, 26, 25)

    ci = jax.lax.broadcasted_iota(jnp.int32, (_C, B), 0)
    W = (w0 * (ci == col0).astype(f32)
         + (ci == col1).astype(f32)
         + (ci == col2).astype(f32))

    def lz(start, width):
        xg = jax.lax.slice_in_dim(x, start, start + width, axis=1)
        m = jnp.max(xg, axis=1, keepdims=True)
        s = jnp.sum(jnp.exp(xg - m), axis=1, keepdims=True)
        return m + jnp.log(s)

    LZ = jnp.concatenate([lz(1, 14), lz(15, 10), lz(25, 2)], axis=1)

    ones = jnp.ones_like(w0)
    W3 = jnp.concatenate([w0, ones, ones], axis=0)
    T = jnp.dot(W3, LZ, preferred_element_type=f32)
    D = jnp.dot(W, x, preferred_element_type=f32)

    di = jax.lax.broadcasted_iota(jnp.int32, (_C, _C), 0)
    dj = jax.lax.broadcasted_iota(jnp.int32, (_C, _C), 1)
    eye = di == dj

    def dsum(start, width):
        m = eye & (di >= start) & (di < start + width)
        return jnp.sum(jnp.where(m, D, 0.0), keepdims=True)

    c0 = T[0:1, 0:1] - dsum(1, 14)
    c1 = T[1:2, 1:2] - dsum(15, 10)
    c2 = T[2:3, 2:3] - dsum(25, 2)
    c22 = jnp.sum(jnp.where(lb == 22, 1.0, 0.0), keepdims=True)
    f0 = jnp.sum(jnp.where(fg0m, 1.0, 0.0), keepdims=True)
    f1 = jnp.sum(jnp.where(fg1m, 1.0, 0.0), keepdims=True)

    ri = jax.lax.broadcasted_iota(jnp.int32, (8, 128), 0)
    cj = jax.lax.broadcasted_iota(jnp.int32, (8, 128), 1)
    tile = jnp.zeros((8, 128), f32)
    for k, v in enumerate((c0, c1, c2, c22, f0, f1)):
        tile = tile + jnp.where((ri == 0) & (cj == k), v, 0.0)

    @pl.when(i == 0)
    def _():
        out_ref[...] = jnp.zeros_like(out_ref)

    out_ref[...] += tile


def _label_maps():
    maps = np.zeros((3, 23), dtype=np.int32)
    maps[:, -1] = -1
    for p, c in enumerate(range(1, 14)):
        maps[0, c] = p + 1
    for p, c in enumerate(range(14, 23)):
        maps[1, c] = p + 1
    for c in range(1, 14):
        maps[2, c] = 0
    for c in range(14, 23):
        maps[2, c] = 1
    return jnp.asarray(maps)


_LMAPS = _label_maps()


def _rare_weight(nl, key):
    n = nl.shape[0]
    fg = nl > 0
    fg_num = jnp.sum(fg)
    bg = nl == 0
    bg_num = jnp.sum(bg)
    bs = fg_num * 8
    full = jnp.logical_or(fg, bg)
    rank = jax.random.permutation(key, n)
    sk = jnp.where(bg, rank, rank + n)
    order = jnp.argsort(sk)
    keep = jnp.zeros((n,), bool).at[order].set(jnp.arange(n) < bs)
    sampled = jnp.logical_or(fg, jnp.logical_and(bg, keep))
    w = jnp.where(bs >= bg_num, full, sampled)
    w = jnp.where(fg_num == 0, jnp.zeros((n,), bool), w)
    return w.astype(jnp.float32)


def _rare_total(cls_score, label):
    key = jax.random.key(1234)
    total = jnp.float32(0.0)
    for g, (start, width) in enumerate(_GROUPS):
        pred = cls_score[:, start:start + width]
        nl = jnp.take(_LMAPS[g], label)
        if g == 2:
            w = (nl != -1).astype(jnp.float32)
        else:
            w = _rare_weight(nl, jax.random.fold_in(key, g))
        avg = jnp.maximum(jnp.sum(w), 1.0)
        logp = jax.nn.log_softmax(pred, axis=1)
        safe = jnp.where(nl == -1, 0, nl)
        loss = -jnp.take_along_axis(logp, safe[:, None], axis=1)[:, 0]
        loss = jnp.where(nl == -1, 0.0, loss)
        total = total + jnp.sum(loss * w) / avg
    return total


def kernel(cls_score, label):
    N = cls_score.shape[0]
    assert N % _B == 0, (N, _B)
    lb3 = label.reshape(N // _B, 1, _B)
    out = pl.pallas_call(
        _partials_body,
        grid=(N // _B,),
        in_specs=[
            pl.BlockSpec((_B, _C), lambda i: (i, 0)),
            pl.BlockSpec((1, 1, _B), lambda i: (i, 0, 0)),
        ],
        out_specs=pl.BlockSpec((8, 128), lambda i: (0, 0)),
        out_shape=jax.ShapeDtypeStruct((8, 128), jnp.float32),
    )(cls_score, lb3)

    S0, S1, S2, C22, F0, F1 = (out[0, k] for k in range(6))
    Nf = jnp.float32(N)
    avg0 = jnp.maximum(Nf - C22, 1.0)
    fast = (jnp.where(F0 > 0, S0 / avg0, 0.0)
            + jnp.where(F1 > 0, S1 / Nf, 0.0)
            + S2 / Nf)
    bg0 = Nf - F0 - C22
    bg1 = Nf - F1
    need = (((F0 * 8 < bg0) & (F0 > 0)) | ((F1 * 8 < bg1) & (F1 > 0)))
    return jax.lax.cond(need,
                        lambda: _rare_total(cls_score, label),
                        lambda: fast)
